# baseline (speedup 1.0000x reference)
import functools

import jax
import jax.numpy as jnp
from jax import lax
from jax.experimental import pallas as pl
from jax.experimental.pallas import tpu as pltpu
from jax.experimental.pallas import tpu_sc as plsc

N = 50000
E = 625000
D = 128
W = 32
NCOL = D // W
NPASS = NCOL + 1

NSC = 2
NTILE = 16
CH = 256
IBLK = 6
NBLK = 26
NCHUNK = IBLK * NBLK
TE = NCHUNK * CH
E_PAD = NTILE * TE
ROWS2D = NSC * E_PAD // CH
R = 50048
TRASH = N
ROWS_PER_TILE = R // NTILE

CPS = 1
SUB = IBLK // CPS
RING = 3


OUT_ROWS_PER_TILE = N // NTILE


def _sc_scatter_build():
    mesh = plsc.VectorSubcoreMesh(core_axis_name="c", subcore_axis_name="s")
    out_type = [jax.ShapeDtypeStruct((NSC * N, W), jnp.float32)
                for _ in range(NPASS)]
    scratch = (
        [pltpu.VMEM((IBLK, CH), jnp.int32) for _ in range(4)]
        + [pltpu.VMEM((CPS * CH, W), jnp.float32)
           for _ in range(RING)]
        + [pltpu.VMEM_SHARED((R, W), jnp.float32)]
        + [pltpu.SemaphoreType.DMA for _ in range(RING + 1)]
    )

    @functools.partial(
        pl.kernel, out_type=out_type, mesh=mesh, scratch_types=scratch,
        compiler_params=pltpu.CompilerParams(use_tc_tiling_on_sc=False))
    def sc_scatter(xs0, xs1, xs2, xs3, srccat, dstcat, zeros,
                   o0, o1, o2, o3, ocnt,
                   s0, s1, d0, d1, m0, m1, m2, acc,
                   isem, g0, g1, g2):
        cid = lax.axis_index("c")
        sid = lax.axis_index("s")
        row0 = cid * (E_PAD // CH) + sid * NCHUNK

        sbufs = [s0, s1]
        dbufs = [d0, d1]
        msgs = [m0, m1, m2]
        gsems = [g0, g1, g2]
        tables = [xs0, xs1, xs2, xs3]
        outs = [o0, o1, o2, o3, ocnt]
        span = pl.ds(sid * ROWS_PER_TILE, ROWS_PER_TILE)

        def src_rows(i):
            return srccat.at[pl.ds(row0 + i * IBLK, IBLK)]

        def dst_rows(i):
            return dstcat.at[pl.ds(row0 + i * IBLK, IBLK)]

        def run_pass(col, is_count):
            table = tables[col]

            def fire(idxbuf, t, r):
                pltpu.async_copy(table.at[idxbuf.at[t]], msgs[r], gsems[r])

            def wait_gather(idxbuf, r):
                pltpu.make_async_copy(
                    table.at[idxbuf.at[0]], msgs[r], gsems[r]).wait()

            def wait_idx(i):
                @pl.when(i < NBLK - 1)
                def _():
                    pltpu.make_async_copy(src_rows(i + 1), s0, isem).wait()
                    pltpu.make_async_copy(dst_rows(i + 1), d0, isem).wait()

            def block(i, b):
                sb, db = sbufs[b], dbufs[b]
                nb = 1 - b
                if is_count:
                    descs = [pltpu.async_copy(
                        msgs[0], acc.at[db.at[t]], gsems[1 + (t % (RING - 1))],
                        add=True) for t in range(SUB)]
                    for dsc in descs:
                        dsc.wait()
                    wait_idx(i)
                else:
                    for t in range(SUB):
                        r = t % RING
                        wait_gather(sb, r)
                        pltpu.sync_copy(msgs[r], acc.at[db.at[t]], add=True)
                        if t + RING < SUB:
                            fire(sb, t + RING, r)
                        else:
                            if t + RING == SUB:
                                wait_idx(i)

                            @pl.when(i < NBLK - 1)
                            def _():
                                fire(sbufs[nb], t + RING - SUB, r)

                @pl.when(i + 2 < NBLK)
                def _():
                    pltpu.async_copy(src_rows(i + 2), sb, isem)
                    pltpu.async_copy(dst_rows(i + 2), db, isem)

            pltpu.sync_copy(src_rows(0), s0)
            pltpu.sync_copy(dst_rows(0), d0)
            pltpu.async_copy(src_rows(1), s1, isem)
            pltpu.async_copy(dst_rows(1), d1, isem)
            if is_count:
                def ob(i, _):
                    msgs[0][i, pl.ds(0, 16)] = jnp.full((16,), 1.0,
                                                        jnp.float32)
                    msgs[0][i, pl.ds(16, 16)] = jnp.full((16,), 1.0,
                                                         jnp.float32)
                    return 0
                lax.fori_loop(0, CPS * CH, ob, 0)
            else:
                for r in range(RING):
                    fire(s0, r, r)

            def loop(it, _):
                block(2 * it, 0)
                block(2 * it + 1, 1)
                return 0
            lax.fori_loop(0, NBLK // 2, loop, 0)

        for p in range(NPASS):
            pltpu.sync_copy(zeros.at[span], acc.at[span])
            plsc.subcore_barrier()
            run_pass(min(p, NCOL - 1), p == NCOL)
            plsc.subcore_barrier()
            pltpu.sync_copy(
                acc.at[pl.ds(sid * OUT_ROWS_PER_TILE, OUT_ROWS_PER_TILE)],
                outs[p].at[pl.ds(cid * N + sid * OUT_ROWS_PER_TILE,
                                 OUT_ROWS_PER_TILE)])
            plsc.subcore_barrier()

    return sc_scatter


_sc_scatter = _sc_scatter_build()

TC_BLK = 1000


def _tc_base_body(xd, wr, b, out):
    out[...] = jnp.dot(xd[...], wr[...],
                       preferred_element_type=jnp.float32) + b[...]


def _tc_body(s0, s1, s2, s3, cnt, base, wl, out):
    aggr = jnp.concatenate([s0[...], s1[...], s2[...], s3[...]], axis=1)
    c = jnp.maximum(cnt[:, 0:1], 1.0)
    aggr = aggr / c
    out[...] = (jnp.dot(aggr, wl[...], preferred_element_type=jnp.float32)
                + base[...])


def _tc_base(x_dst, Wr, bl):
    nblk = N // TC_BLK
    full = pl.BlockSpec((TC_BLK, D), lambda i: (i, 0))
    wspec = pl.BlockSpec((D, D), lambda i: (0, 0))
    bspec = pl.BlockSpec((1, D), lambda i: (0, 0))
    return pl.pallas_call(
        _tc_base_body,
        grid=(nblk,),
        in_specs=[full, wspec, bspec],
        out_specs=full,
        out_shape=jax.ShapeDtypeStruct((N, D), jnp.float32),
    )(x_dst, Wr, bl.reshape(1, D))


def _tc_final(s0, s1, s2, s3, cnt, base, Wl, rel):
    nblk = N // TC_BLK
    off = rel * nblk
    col = pl.BlockSpec((TC_BLK, W), lambda i: (i + off, 0))
    full = pl.BlockSpec((TC_BLK, D), lambda i: (i, 0))
    wspec = pl.BlockSpec((D, D), lambda i: (0, 0))
    return pl.pallas_call(
        _tc_body,
        grid=(nblk,),
        in_specs=[col, col, col, col, col, full, wspec],
        out_specs=full,
        out_shape=jax.ShapeDtypeStruct((N, D), jnp.float32),
    )(s0, s1, s2, s3, cnt, base, Wl)


def kernel(x_user, x_item, edge_index_user_rates_item,
           edge_index_item_rated_by_user,
           Wl_rates, bl_rates, Wr_rates,
           Wl_rev, bl_rev, Wr_rev):
    src0 = edge_index_user_rates_item[0].astype(jnp.int32)
    dst0 = edge_index_user_rates_item[1].astype(jnp.int32)
    src1 = edge_index_item_rated_by_user[0].astype(jnp.int32) + N
    dst1 = edge_index_item_rated_by_user[1].astype(jnp.int32)

    zpad = jnp.zeros((E_PAD - E,), jnp.int32)
    tpad = jnp.full((E_PAD - E,), TRASH, jnp.int32)
    srccat = jnp.concatenate([src0, zpad, src1, zpad]).reshape(ROWS2D, CH)
    dstcat = jnp.concatenate([dst0, tpad, dst1, tpad]).reshape(ROWS2D, CH)
    xs = [jnp.concatenate([x_user[:, c * W:(c + 1) * W],
                           x_item[:, c * W:(c + 1) * W]], axis=0)
          for c in range(NCOL)]
    zeros = jnp.zeros((R, W), jnp.float32)

    base_item = _tc_base(x_item, Wr_rates, bl_rates)
    base_user = _tc_base(x_user, Wr_rev, bl_rev)

    o0, o1, o2, o3, ocnt = _sc_scatter(xs[0], xs[1], xs[2], xs[3],
                                       srccat, dstcat, zeros)

    out_item = _tc_final(o0, o1, o2, o3, ocnt, base_item, Wl_rates, 0)
    out_user = _tc_final(o0, o1, o2, o3, ocnt, base_user, Wl_rev, 1)
    return (out_user, out_item)


# speedup vs baseline: 1.0087x; 1.0087x over previous
import functools

import jax
import jax.numpy as jnp
from jax import lax
from jax.experimental import pallas as pl
from jax.experimental.pallas import tpu as pltpu
from jax.experimental.pallas import tpu_sc as plsc

N = 50000
E = 625000
D = 128
W = 32
NCOL = D // W
NPASS = NCOL + 1

NSC = 2
NTILE = 16
CH = 256
IBLK = 6
NBLK = 26
NCHUNK = IBLK * NBLK
TE = NCHUNK * CH
E_PAD = NTILE * TE
ROWS2D = NSC * E_PAD // CH
R = 50048
TRASH = N
ROWS_PER_TILE = R // NTILE

CPS = 1
SUB = IBLK // CPS
RING = 3


OUT_ROWS_PER_TILE = N // NTILE


def _sc_scatter_build():
    mesh = plsc.VectorSubcoreMesh(core_axis_name="c", subcore_axis_name="s")
    out_type = [jax.ShapeDtypeStruct((NSC * N, W), jnp.float32)
                for _ in range(NPASS)]
    scratch = (
        [pltpu.VMEM((IBLK, CH), jnp.int32) for _ in range(4)]
        + [pltpu.VMEM((CPS * CH, W), jnp.float32)
           for _ in range(RING)]
        + [pltpu.VMEM_SHARED((R, W), jnp.float32)]
        + [pltpu.SemaphoreType.DMA for _ in range(RING + 1)]
    )

    @functools.partial(
        pl.kernel, out_type=out_type, mesh=mesh, scratch_types=scratch,
        compiler_params=pltpu.CompilerParams(use_tc_tiling_on_sc=False))
    def sc_scatter(xs0, xs1, xs2, xs3, srccat, dstcat, zeros,
                   o0, o1, o2, o3, ocnt,
                   s0, s1, d0, d1, m0, m1, m2, acc,
                   isem, g0, g1, g2):
        cid = lax.axis_index("c")
        sid = lax.axis_index("s")
        row0 = cid * (E_PAD // CH) + sid * NCHUNK

        sbufs = [s0, s1]
        dbufs = [d0, d1]
        msgs = [m0, m1, m2]
        gsems = [g0, g1, g2]
        tables = [xs0, xs1, xs2, xs3]
        outs = [o0, o1, o2, o3, ocnt]
        span = pl.ds(sid * ROWS_PER_TILE, ROWS_PER_TILE)

        def src_rows(i):
            return srccat.at[pl.ds(row0 + i * IBLK, IBLK)]

        def dst_rows(i):
            return dstcat.at[pl.ds(row0 + i * IBLK, IBLK)]

        def run_pass(col, is_count):
            table = tables[col]

            def fire(idxbuf, t, r):
                pltpu.async_copy(table.at[idxbuf.at[t]], msgs[r], gsems[r])

            def wait_gather(idxbuf, r):
                pltpu.make_async_copy(
                    table.at[idxbuf.at[0]], msgs[r], gsems[r]).wait()

            def wait_idx(i):
                @pl.when(i < NBLK - 1)
                def _():
                    pltpu.make_async_copy(src_rows(i + 1), s0, isem).wait()
                    pltpu.make_async_copy(dst_rows(i + 1), d0, isem).wait()

            def block(i, b):
                sb, db = sbufs[b], dbufs[b]
                nb = 1 - b
                if is_count:
                    descs = [pltpu.async_copy(
                        msgs[0], acc.at[db.at[t]], gsems[1 + (t % (RING - 1))],
                        add=True) for t in range(SUB)]
                    for dsc in descs:
                        dsc.wait()
                    wait_idx(i)
                else:
                    for t in range(SUB):
                        r = t % RING
                        wait_gather(sb, r)
                        pltpu.sync_copy(msgs[r], acc.at[db.at[t]], add=True)
                        if t + RING < SUB:
                            fire(sb, t + RING, r)
                        else:
                            if t + RING == SUB:
                                wait_idx(i)

                            @pl.when(i < NBLK - 1)
                            def _():
                                fire(sbufs[nb], t + RING - SUB, r)

                @pl.when(i + 2 < NBLK)
                def _():
                    pltpu.async_copy(src_rows(i + 2), sb, isem)
                    pltpu.async_copy(dst_rows(i + 2), db, isem)

            pltpu.sync_copy(src_rows(0), s0)
            pltpu.sync_copy(dst_rows(0), d0)
            pltpu.async_copy(src_rows(1), s1, isem)
            pltpu.async_copy(dst_rows(1), d1, isem)
            if is_count:
                def ob(i, _):
                    msgs[0][i, pl.ds(0, 16)] = jnp.full((16,), 1.0,
                                                        jnp.float32)
                    msgs[0][i, pl.ds(16, 16)] = jnp.full((16,), 1.0,
                                                         jnp.float32)
                    return 0
                lax.fori_loop(0, CPS * CH, ob, 0)
            else:
                for r in range(RING):
                    fire(s0, r, r)

            def loop(it, _):
                block(2 * it, 0)
                block(2 * it + 1, 1)
                return 0
            lax.fori_loop(0, NBLK // 2, loop, 0)

        for p in range(NPASS):
            pltpu.sync_copy(zeros.at[span], acc.at[span])
            plsc.subcore_barrier()
            run_pass(min(p, NCOL - 1), p == NCOL)
            plsc.subcore_barrier()
            pltpu.sync_copy(
                acc.at[pl.ds(sid * OUT_ROWS_PER_TILE, OUT_ROWS_PER_TILE)],
                outs[p].at[pl.ds(cid * N + sid * OUT_ROWS_PER_TILE,
                                 OUT_ROWS_PER_TILE)])
            plsc.subcore_barrier()

    return sc_scatter


_sc_scatter = _sc_scatter_build()

TC_BLK = 1000


def _tc_body(s0, s1, s2, s3, cnt, xd, wl, wr, b, out):
    aggr = jnp.concatenate([s0[...], s1[...], s2[...], s3[...]], axis=1)
    c = jnp.maximum(cnt[:, 0:1], 1.0)
    aggr = aggr / c
    out[...] = (jnp.dot(aggr, wl[...], preferred_element_type=jnp.float32)
                + jnp.dot(xd[...], wr[...],
                          preferred_element_type=jnp.float32)
                + b[...])


def _tc_final(s0, s1, s2, s3, cnt, x_dst, Wl, Wr, bl, rel):
    nblk = N // TC_BLK
    off = rel * nblk
    col = pl.BlockSpec((TC_BLK, W), lambda i: (i + off, 0))
    full = pl.BlockSpec((TC_BLK, D), lambda i: (i, 0))
    wspec = pl.BlockSpec((D, D), lambda i: (0, 0))
    bspec = pl.BlockSpec((1, D), lambda i: (0, 0))
    return pl.pallas_call(
        _tc_body,
        grid=(nblk,),
        in_specs=[col, col, col, col, col, full, wspec, wspec, bspec],
        out_specs=full,
        out_shape=jax.ShapeDtypeStruct((N, D), jnp.float32),
    )(s0, s1, s2, s3, cnt, x_dst, Wl, Wr, bl.reshape(1, D))


def kernel(x_user, x_item, edge_index_user_rates_item,
           edge_index_item_rated_by_user,
           Wl_rates, bl_rates, Wr_rates,
           Wl_rev, bl_rev, Wr_rev):
    src0 = edge_index_user_rates_item[0].astype(jnp.int32)
    dst0 = edge_index_user_rates_item[1].astype(jnp.int32)
    src1 = edge_index_item_rated_by_user[0].astype(jnp.int32) + N
    dst1 = edge_index_item_rated_by_user[1].astype(jnp.int32)

    zpad = jnp.zeros((E_PAD - E,), jnp.int32)
    tpad = jnp.full((E_PAD - E,), TRASH, jnp.int32)
    srccat = jnp.concatenate([src0, zpad, src1, zpad]).reshape(ROWS2D, CH)
    dstcat = jnp.concatenate([dst0, tpad, dst1, tpad]).reshape(ROWS2D, CH)
    xs = [jnp.concatenate([x_user[:, c * W:(c + 1) * W],
                           x_item[:, c * W:(c + 1) * W]], axis=0)
          for c in range(NCOL)]
    zeros = jnp.zeros((R, W), jnp.float32)

    o0, o1, o2, o3, ocnt = _sc_scatter(xs[0], xs[1], xs[2], xs[3],
                                       srccat, dstcat, zeros)

    out_item = _tc_final(o0, o1, o2, o3, ocnt,
                         x_item, Wl_rates, Wr_rates, bl_rates, 0)
    out_user = _tc_final(o0, o1, o2, o3, ocnt,
                         x_user, Wl_rev, Wr_rev, bl_rev, 1)
    return (out_user, out_item)
